# Initial kernel scaffold; baseline (speedup 1.0000x reference)
import jax
import jax.numpy as jnp
from jax.experimental import pallas as pl


def kernel(x, edge_index, W1, b1, Wmu, bmu, Wls, bls):
    raise NotImplementedError("write your pallas kernel here")


# speedup vs baseline: 54.4365x; 54.4365x over previous
import functools
import jax
import jax.numpy as jnp
from jax import lax
from jax.experimental import pallas as pl
from jax.experimental.pallas import tpu as pltpu
from jax.experimental.pallas import tpu_sc as plsc

N = 10000
E = 320000
D_IN = 128
D_HID = 20
D_OUT = 10
DH = 32

NC, NS = 2, 16
NW = NC * NS
EPW = E // NW
CHUNK = 125
NCHUNK = EPW // CHUNK
ROWS_PER_SUB = N // NS

_mesh = plsc.VectorSubcoreMesh(
    core_axis_name="c", subcore_axis_name="s", num_cores=NC, num_subcores=NS
)


@functools.partial(
    pl.kernel,
    out_type=jax.ShapeDtypeStruct((NC * N,), jnp.float32),
    mesh=_mesh,
    scratch_types=[
        pltpu.VMEM((NCHUNK, CHUNK), jnp.int32),
        pltpu.VMEM((CHUNK,), jnp.float32),
        pltpu.VMEM((624,), jnp.float32),
        pltpu.VMEM_SHARED((N,), jnp.float32),
    ],
)
def _deg_kernel(col_hbm, ones_hbm, zeros_hbm, out_hbm, colv, onesv, zbuf, acc):
    c = lax.axis_index("c")
    s = lax.axis_index("s")
    M = 624
    REM = N - NS * M
    sl = pl.ds(s * M, M)
    rem = pl.ds(NS * M, REM)
    pltpu.sync_copy(zeros_hbm, zbuf)
    pltpu.sync_copy(zbuf, acc.at[sl])

    @pl.when(s == 0)
    def _():
        pltpu.sync_copy(zbuf.at[pl.ds(0, REM)], acc.at[rem])

    pltpu.sync_copy(col_hbm.at[c, s], colv)
    pltpu.sync_copy(ones_hbm, onesv)
    plsc.subcore_barrier()

    def body(j, carry):
        pltpu.sync_copy(onesv, acc.at[colv.at[j]], add=True)
        return carry

    lax.fori_loop(0, NCHUNK, body, 0)
    plsc.subcore_barrier()
    pltpu.sync_copy(acc.at[sl], zbuf)
    pltpu.sync_copy(zbuf, out_hbm.at[pl.ds(c * N + s * M, M)])

    @pl.when(s == 0)
    def _():
        pltpu.sync_copy(acc.at[rem], zbuf.at[pl.ds(0, REM)])
        pltpu.sync_copy(zbuf.at[pl.ds(0, REM)], out_hbm.at[pl.ds(c * N + NS * M, REM)])


@functools.partial(
    pl.kernel,
    out_type=jax.ShapeDtypeStruct((NC, N, DH), jnp.float32),
    mesh=_mesh,
    scratch_types=[
        pltpu.VMEM((NCHUNK, CHUNK), jnp.int32),
        pltpu.VMEM((NCHUNK, CHUNK), jnp.int32),
        pltpu.VMEM((CHUNK, DH), jnp.float32),
        pltpu.VMEM((CHUNK, DH), jnp.float32),
        pltpu.VMEM((624, DH), jnp.float32),
        pltpu.VMEM_SHARED((N, DH), jnp.float32),
        pltpu.SemaphoreType.DMA,
        pltpu.SemaphoreType.DMA,
    ],
    compiler_params=pltpu.CompilerParams(use_tc_tiling_on_sc=False),
)
def _agg_kernel(hs_hbm, row_hbm, col_hbm, out_hbm,
                rowv, colv, bufa, bufb, sbuf, acc, sema, semb):
    c = lax.axis_index("c")
    s = lax.axis_index("s")
    M = 624
    REM = N - NS * M
    sl = pl.ds(s * M, M)
    rem = pl.ds(NS * M, REM)
    pltpu.sync_copy(hs_hbm.at[sl], sbuf)
    pltpu.sync_copy(sbuf, acc.at[sl])

    @pl.when(s == 0)
    def _():
        pltpu.sync_copy(hs_hbm.at[rem], sbuf.at[pl.ds(0, REM)])
        pltpu.sync_copy(sbuf.at[pl.ds(0, REM)], acc.at[rem])

    pltpu.sync_copy(row_hbm.at[c, s], rowv)
    pltpu.sync_copy(col_hbm.at[c, s], colv)
    plsc.subcore_barrier()

    pltpu.async_copy(hs_hbm.at[rowv.at[0]], bufa, sema)

    def body(jj, carry):
        j0 = jj * 2
        pltpu.async_copy(hs_hbm.at[rowv.at[j0 + 1]], bufb, semb)
        pltpu.make_async_copy(hs_hbm.at[rowv.at[j0]], bufa, sema).wait()
        pltpu.sync_copy(bufa, acc.at[colv.at[j0]], add=True)

        @pl.when(jj < NCHUNK // 2 - 1)
        def _():
            pltpu.async_copy(hs_hbm.at[rowv.at[j0 + 2]], bufa, sema)

        pltpu.make_async_copy(hs_hbm.at[rowv.at[j0 + 1]], bufb, semb).wait()
        pltpu.sync_copy(bufb, acc.at[colv.at[j0 + 1]], add=True)
        return carry

    lax.fori_loop(0, NCHUNK // 2, body, 0)
    plsc.subcore_barrier()
    pltpu.sync_copy(acc.at[sl], sbuf)
    pltpu.sync_copy(sbuf, out_hbm.at[c, sl])

    @pl.when(s == 0)
    def _():
        pltpu.sync_copy(acc.at[rem], sbuf.at[pl.ds(0, REM)])
        pltpu.sync_copy(sbuf.at[pl.ds(0, REM)], out_hbm.at[c, rem])


def _prep1_body(x_ref, w_ref, dinv_ref, o_ref):
    o_ref[...] = dinv_ref[...] * jnp.dot(
        x_ref[...], w_ref[...], preferred_element_type=jnp.float32)


_prep1 = pl.pallas_call(
    _prep1_body,
    out_shape=jax.ShapeDtypeStruct((N, DH), jnp.float32),
)


def _mid_body(p_ref, hs1_ref, dinv_ref, b_ref, o_ref):
    agg = p_ref[0] + p_ref[1] - hs1_ref[...]
    h = jnp.maximum(dinv_ref[...] * agg + b_ref[...], 0.0)
    o_ref[...] = dinv_ref[...] * h


_mid = pl.pallas_call(
    _mid_body,
    out_shape=jax.ShapeDtypeStruct((N, DH), jnp.float32),
)


def _fin_body(q_ref, hs2_ref, dinv_ref, wmu_ref, bmu_ref, wls_ref, bls_ref,
              mu_ref, ls_ref):
    t = dinv_ref[...] * (q_ref[0] + q_ref[1] - hs2_ref[...])
    mu_ref[...] = jnp.dot(t, wmu_ref[...],
                          preferred_element_type=jnp.float32) + bmu_ref[...]
    ls_ref[...] = jnp.dot(t, wls_ref[...],
                          preferred_element_type=jnp.float32) + bls_ref[...]


_fin = pl.pallas_call(
    _fin_body,
    out_shape=(
        jax.ShapeDtypeStruct((N, D_OUT), jnp.float32),
        jax.ShapeDtypeStruct((N, D_OUT), jnp.float32),
    ),
)


def kernel(x, edge_index, W1, b1, Wmu, bmu, Wls, bls):
    row = edge_index[0].reshape(NC, NS, NCHUNK, CHUNK)
    col = edge_index[1].reshape(NC, NS, NCHUNK, CHUNK)
    ones = jnp.ones((CHUNK,), jnp.float32)
    zeros = jnp.zeros((624,), jnp.float32)

    degp = _deg_kernel(col, ones, zeros).reshape(NC, N)
    dinv = lax.rsqrt(degp[0] + degp[1] + 1.0).reshape(N, 1)

    W1p = jnp.zeros((D_IN, DH), jnp.float32).at[:, :D_HID].set(W1)
    b1p = jnp.zeros((1, DH), jnp.float32).at[0, :D_HID].set(b1)
    Wmup = jnp.zeros((DH, D_OUT), jnp.float32).at[:D_HID].set(Wmu)
    Wlsp = jnp.zeros((DH, D_OUT), jnp.float32).at[:D_HID].set(Wls)

    hs1 = _prep1(x, W1p, dinv)
    p = _agg_kernel(hs1, row, col)
    hs2 = _mid(p, hs1, dinv, b1p)
    q = _agg_kernel(hs2, row, col)
    mu, ls = _fin(q, hs2, dinv, Wmup, bmu.reshape(1, D_OUT),
                  Wlsp, bls.reshape(1, D_OUT))
    return (mu, ls)


# speedup vs baseline: 55.8340x; 1.0257x over previous
import functools
import jax
import jax.numpy as jnp
from jax import lax
from jax.experimental import pallas as pl
from jax.experimental.pallas import tpu as pltpu
from jax.experimental.pallas import tpu_sc as plsc

N = 10000
E = 320000
D_IN = 128
D_HID = 20
D_OUT = 10
DH = 32

NC, NS = 2, 16
NW = NC * NS
EPW = E // NW
CHUNK = 125
NCHUNK = EPW // CHUNK
ROWS_PER_SUB = N // NS

_mesh = plsc.VectorSubcoreMesh(
    core_axis_name="c", subcore_axis_name="s", num_cores=NC, num_subcores=NS
)


@functools.partial(
    pl.kernel,
    out_type=jax.ShapeDtypeStruct((NC * N,), jnp.float32),
    mesh=_mesh,
    scratch_types=[
        pltpu.VMEM((NCHUNK, CHUNK), jnp.int32),
        pltpu.VMEM((CHUNK,), jnp.float32),
        pltpu.VMEM((624,), jnp.float32),
        pltpu.VMEM_SHARED((N,), jnp.float32),
    ],
)
def _deg_kernel(col_hbm, ones_hbm, zeros_hbm, out_hbm, colv, onesv, zbuf, acc):
    c = lax.axis_index("c")
    s = lax.axis_index("s")
    M = 624
    REM = N - NS * M
    sl = pl.ds(s * M, M)
    rem = pl.ds(NS * M, REM)
    pltpu.sync_copy(zeros_hbm, zbuf)
    pltpu.sync_copy(zbuf, acc.at[sl])

    @pl.when(s == 0)
    def _():
        pltpu.sync_copy(zbuf.at[pl.ds(0, REM)], acc.at[rem])

    pltpu.sync_copy(col_hbm.at[c, s], colv)
    pltpu.sync_copy(ones_hbm, onesv)
    plsc.subcore_barrier()

    def body(j, carry):
        pltpu.sync_copy(onesv, acc.at[colv.at[j]], add=True)
        return carry

    lax.fori_loop(0, NCHUNK, body, 0)
    plsc.subcore_barrier()
    pltpu.sync_copy(acc.at[sl], zbuf)
    pltpu.sync_copy(zbuf, out_hbm.at[pl.ds(c * N + s * M, M)])

    @pl.when(s == 0)
    def _():
        pltpu.sync_copy(acc.at[rem], zbuf.at[pl.ds(0, REM)])
        pltpu.sync_copy(zbuf.at[pl.ds(0, REM)], out_hbm.at[pl.ds(c * N + NS * M, REM)])


@functools.partial(
    pl.kernel,
    out_type=jax.ShapeDtypeStruct((NC, N, DH), jnp.float32),
    mesh=_mesh,
    scratch_types=[
        pltpu.VMEM((NCHUNK, CHUNK), jnp.int32),
        pltpu.VMEM((NCHUNK, CHUNK), jnp.int32),
        pltpu.VMEM((CHUNK, DH), jnp.float32),
        pltpu.VMEM((CHUNK, DH), jnp.float32),
        pltpu.VMEM((624, DH), jnp.float32),
        pltpu.VMEM_SHARED((N, DH), jnp.float32),
        pltpu.SemaphoreType.DMA,
        pltpu.SemaphoreType.DMA,
    ],
    compiler_params=pltpu.CompilerParams(use_tc_tiling_on_sc=False),
)
def _agg_kernel(hs_hbm, row_hbm, col_hbm, out_hbm,
                rowv, colv, bufa, bufb, sbuf, acc, sema, semb):
    c = lax.axis_index("c")
    s = lax.axis_index("s")
    M = 624
    REM = N - NS * M
    sl = pl.ds(s * M, M)
    rem = pl.ds(NS * M, REM)
    pltpu.sync_copy(hs_hbm.at[sl], sbuf)
    pltpu.sync_copy(sbuf, acc.at[sl])

    @pl.when(s == 0)
    def _():
        pltpu.sync_copy(hs_hbm.at[rem], sbuf.at[pl.ds(0, REM)])
        pltpu.sync_copy(sbuf.at[pl.ds(0, REM)], acc.at[rem])

    pltpu.sync_copy(row_hbm.at[c, s], rowv)
    pltpu.sync_copy(col_hbm.at[c, s], colv)
    plsc.subcore_barrier()

    pltpu.async_copy(hs_hbm.at[rowv.at[0]], bufa, sema)

    def body(jj, carry):
        j0 = jj * 2
        pltpu.async_copy(hs_hbm.at[rowv.at[j0 + 1]], bufb, semb)
        pltpu.make_async_copy(hs_hbm.at[rowv.at[j0]], bufa, sema).wait()
        pltpu.sync_copy(bufa, acc.at[colv.at[j0]], add=True)

        @pl.when(jj < NCHUNK // 2 - 1)
        def _():
            pltpu.async_copy(hs_hbm.at[rowv.at[j0 + 2]], bufa, sema)

        pltpu.make_async_copy(hs_hbm.at[rowv.at[j0 + 1]], bufb, semb).wait()
        pltpu.sync_copy(bufb, acc.at[colv.at[j0 + 1]], add=True)
        return carry

    lax.fori_loop(0, NCHUNK // 2, body, 0)
    plsc.subcore_barrier()
    pltpu.sync_copy(acc.at[sl], sbuf)
    pltpu.sync_copy(sbuf, out_hbm.at[c, sl])

    @pl.when(s == 0)
    def _():
        pltpu.sync_copy(acc.at[rem], sbuf.at[pl.ds(0, REM)])
        pltpu.sync_copy(sbuf.at[pl.ds(0, REM)], out_hbm.at[c, rem])


def _mm1_body(x_ref, w_ref, o_ref):
    w = jnp.concatenate(
        [w_ref[...], jnp.zeros((D_IN, DH - D_HID), jnp.float32)], axis=1)
    o_ref[...] = jnp.dot(x_ref[...], w, preferred_element_type=jnp.float32)


_mm1 = pl.pallas_call(
    _mm1_body,
    out_shape=jax.ShapeDtypeStruct((N, DH), jnp.float32),
)


def _scale_body(deg_ref, y_ref, hs_ref, dinv_ref):
    d = lax.rsqrt(deg_ref[0] + deg_ref[1] + 1.0).reshape(N, 1)
    dinv_ref[...] = d
    hs_ref[...] = d * y_ref[...]


_scale = pl.pallas_call(
    _scale_body,
    out_shape=(
        jax.ShapeDtypeStruct((N, DH), jnp.float32),
        jax.ShapeDtypeStruct((N, 1), jnp.float32),
    ),
)


def _mid_body(p_ref, hs1_ref, dinv_ref, b_ref, o_ref):
    b = jnp.concatenate(
        [b_ref[...], jnp.zeros((1, DH - D_HID), jnp.float32)], axis=1)
    agg = p_ref[0] + p_ref[1] - hs1_ref[...]
    h = jnp.maximum(dinv_ref[...] * agg + b, 0.0)
    o_ref[...] = dinv_ref[...] * h


_mid = pl.pallas_call(
    _mid_body,
    out_shape=jax.ShapeDtypeStruct((N, DH), jnp.float32),
)


def _fin_body(q_ref, hs2_ref, dinv_ref, wmu_ref, bmu_ref, wls_ref, bls_ref,
              mu_ref, ls_ref):
    zpad = jnp.zeros((DH - D_HID, D_OUT), jnp.float32)
    wmu = jnp.concatenate([wmu_ref[...], zpad], axis=0)
    wls = jnp.concatenate([wls_ref[...], zpad], axis=0)
    t = dinv_ref[...] * (q_ref[0] + q_ref[1] - hs2_ref[...])
    mu_ref[...] = jnp.dot(t, wmu,
                          preferred_element_type=jnp.float32) + bmu_ref[...]
    ls_ref[...] = jnp.dot(t, wls,
                          preferred_element_type=jnp.float32) + bls_ref[...]


_fin = pl.pallas_call(
    _fin_body,
    out_shape=(
        jax.ShapeDtypeStruct((N, D_OUT), jnp.float32),
        jax.ShapeDtypeStruct((N, D_OUT), jnp.float32),
    ),
)


def kernel(x, edge_index, W1, b1, Wmu, bmu, Wls, bls):
    row = edge_index[0].reshape(NC, NS, NCHUNK, CHUNK)
    col = edge_index[1].reshape(NC, NS, NCHUNK, CHUNK)
    ones = jnp.ones((CHUNK,), jnp.float32)
    zeros = jnp.zeros((624,), jnp.float32)

    degp = _deg_kernel(col, ones, zeros).reshape(NC, N)
    y1 = _mm1(x, W1)
    hs1, dinv = _scale(degp, y1)

    p = _agg_kernel(hs1, row, col)
    hs2 = _mid(p, hs1, dinv, b1.reshape(1, D_HID))
    q = _agg_kernel(hs2, row, col)
    mu, ls = _fin(q, hs2, dinv, Wmu, bmu.reshape(1, D_OUT),
                  Wls, bls.reshape(1, D_OUT))
    return (mu, ls)


# speedup vs baseline: 60.0176x; 1.0749x over previous
import functools
import jax
import jax.numpy as jnp
from jax import lax
from jax.experimental import pallas as pl
from jax.experimental.pallas import tpu as pltpu
from jax.experimental.pallas import tpu_sc as plsc

N = 10000
E = 320000
D_IN = 128
D_HID = 20
D_OUT = 10
DH = 32

NC, NS = 2, 16
NW = NC * NS
EPW = E // NW
CHUNK = 125
NCHUNK = EPW // CHUNK
ROWS_PER_SUB = N // NS

_mesh = plsc.VectorSubcoreMesh(
    core_axis_name="c", subcore_axis_name="s", num_cores=NC, num_subcores=NS
)


@functools.partial(
    pl.kernel,
    out_type=jax.ShapeDtypeStruct((NC * N,), jnp.float32),
    mesh=_mesh,
    scratch_types=[
        pltpu.VMEM((NCHUNK, CHUNK), jnp.int32),
        pltpu.VMEM((CHUNK,), jnp.float32),
        pltpu.VMEM((624,), jnp.float32),
        pltpu.VMEM_SHARED((N,), jnp.float32),
    ],
)
def _deg_kernel(col_hbm, ones_hbm, zeros_hbm, out_hbm, colv, onesv, zbuf, acc):
    c = lax.axis_index("c")
    s = lax.axis_index("s")
    M = 624
    REM = N - NS * M
    sl = pl.ds(s * M, M)
    rem = pl.ds(NS * M, REM)
    pltpu.sync_copy(zeros_hbm, zbuf)
    pltpu.sync_copy(zbuf, acc.at[sl])

    @pl.when(s == 0)
    def _():
        pltpu.sync_copy(zbuf.at[pl.ds(0, REM)], acc.at[rem])

    pltpu.sync_copy(col_hbm.at[c, s], colv)
    pltpu.sync_copy(ones_hbm, onesv)
    plsc.subcore_barrier()

    def body(j, carry):
        pltpu.sync_copy(onesv, acc.at[colv.at[j]], add=True)
        return carry

    lax.fori_loop(0, NCHUNK, body, 0)
    plsc.subcore_barrier()
    pltpu.sync_copy(acc.at[sl], zbuf)
    pltpu.sync_copy(zbuf, out_hbm.at[pl.ds(c * N + s * M, M)])

    @pl.when(s == 0)
    def _():
        pltpu.sync_copy(acc.at[rem], zbuf.at[pl.ds(0, REM)])
        pltpu.sync_copy(zbuf.at[pl.ds(0, REM)], out_hbm.at[pl.ds(c * N + NS * M, REM)])


@functools.partial(
    pl.kernel,
    out_type=jax.ShapeDtypeStruct((NC, N, DH), jnp.float32),
    mesh=_mesh,
    scratch_types=[
        pltpu.VMEM((NCHUNK, CHUNK), jnp.int32),
        pltpu.VMEM((NCHUNK, CHUNK), jnp.int32),
        pltpu.VMEM((CHUNK, DH), jnp.float32),
        pltpu.VMEM((CHUNK, DH), jnp.float32),
        pltpu.VMEM((CHUNK, DH), jnp.float32),
        pltpu.VMEM((CHUNK, DH), jnp.float32),
        pltpu.VMEM((624, DH), jnp.float32),
        pltpu.VMEM_SHARED((N, DH), jnp.float32),
        pltpu.SemaphoreType.DMA,
        pltpu.SemaphoreType.DMA,
        pltpu.SemaphoreType.DMA,
        pltpu.SemaphoreType.DMA,
        pltpu.SemaphoreType.DMA,
        pltpu.SemaphoreType.DMA,
        pltpu.SemaphoreType.DMA,
        pltpu.SemaphoreType.DMA,
        pltpu.SemaphoreType.DMA,
        pltpu.SemaphoreType.DMA,
    ],
    compiler_params=pltpu.CompilerParams(use_tc_tiling_on_sc=False),
)
def _agg_kernel(hs_hbm, row_hbm, col_hbm, out_hbm,
                rowv, colv, b0, b1, b2, b3, sbuf, acc,
                g0, g1, g2, g3, s0, s1, s2, s3, ri, ci):
    c = lax.axis_index("c")
    s = lax.axis_index("s")
    M = 624
    REM = N - NS * M
    sl = pl.ds(s * M, M)
    rem = pl.ds(NS * M, REM)
    bufs = (b0, b1, b2, b3)
    gsem = (g0, g1, g2, g3)
    ssem = (s0, s1, s2, s3)
    pltpu.async_copy(row_hbm.at[c, s], rowv, ri)
    pltpu.async_copy(col_hbm.at[c, s], colv, ci)
    pltpu.sync_copy(hs_hbm.at[sl], sbuf)
    pltpu.sync_copy(sbuf, acc.at[sl])

    @pl.when(s == 0)
    def _():
        pltpu.sync_copy(hs_hbm.at[rem], sbuf.at[pl.ds(0, REM)])
        pltpu.sync_copy(sbuf.at[pl.ds(0, REM)], acc.at[rem])

    pltpu.make_async_copy(row_hbm.at[c, s], rowv, ri).wait()
    pltpu.make_async_copy(col_hbm.at[c, s], colv, ci).wait()
    plsc.subcore_barrier()

    pltpu.async_copy(hs_hbm.at[rowv.at[0]], bufs[0], gsem[0])
    pltpu.async_copy(hs_hbm.at[rowv.at[1]], bufs[1], gsem[1])

    def body(jj, carry):
        j0 = jj * 4
        for b in range(4):
            j = j0 + b
            pltpu.make_async_copy(hs_hbm.at[rowv.at[j]], bufs[b], gsem[b]).wait()
            pltpu.async_copy(bufs[b], acc.at[colv.at[j]], ssem[b], add=True)
            m = j + 2
            mb = (b + 2) % 4

            @pl.when(m < NCHUNK)
            def _():
                @pl.when(m >= 4)
                def _():
                    pltpu.make_async_copy(
                        bufs[mb], acc.at[colv.at[m - 4]], ssem[mb]
                    ).wait()

                pltpu.async_copy(hs_hbm.at[rowv.at[m]], bufs[mb], gsem[mb])

        return carry

    lax.fori_loop(0, NCHUNK // 4, body, 0)
    for b in range(4):
        j = NCHUNK - 4 + b
        pltpu.make_async_copy(bufs[b], acc.at[colv.at[j]], ssem[b]).wait()
    plsc.subcore_barrier()
    pltpu.sync_copy(acc.at[sl], sbuf)
    pltpu.sync_copy(sbuf, out_hbm.at[c, sl])

    @pl.when(s == 0)
    def _():
        pltpu.sync_copy(acc.at[rem], sbuf.at[pl.ds(0, REM)])
        pltpu.sync_copy(sbuf.at[pl.ds(0, REM)], out_hbm.at[c, rem])


def _mm1_body(x_ref, w_ref, o_ref):
    w = jnp.concatenate(
        [w_ref[...], jnp.zeros((D_IN, DH - D_HID), jnp.float32)], axis=1)
    o_ref[...] = jnp.dot(x_ref[...], w, preferred_element_type=jnp.float32)


_mm1 = pl.pallas_call(
    _mm1_body,
    out_shape=jax.ShapeDtypeStruct((N, DH), jnp.float32),
)


def _scale_body(deg_ref, y_ref, hs_ref, dinv_ref):
    d = lax.rsqrt(deg_ref[0] + deg_ref[1] + 1.0).reshape(N, 1)
    dinv_ref[...] = d
    hs_ref[...] = d * y_ref[...]


_scale = pl.pallas_call(
    _scale_body,
    out_shape=(
        jax.ShapeDtypeStruct((N, DH), jnp.float32),
        jax.ShapeDtypeStruct((N, 1), jnp.float32),
    ),
)


def _mid_body(p_ref, hs1_ref, dinv_ref, b_ref, o_ref):
    b = jnp.concatenate(
        [b_ref[...], jnp.zeros((1, DH - D_HID), jnp.float32)], axis=1)
    agg = p_ref[0] + p_ref[1] - hs1_ref[...]
    h = jnp.maximum(dinv_ref[...] * agg + b, 0.0)
    o_ref[...] = dinv_ref[...] * h


_mid = pl.pallas_call(
    _mid_body,
    out_shape=jax.ShapeDtypeStruct((N, DH), jnp.float32),
)


def _fin_body(q_ref, hs2_ref, dinv_ref, wmu_ref, bmu_ref, wls_ref, bls_ref,
              mu_ref, ls_ref):
    zpad = jnp.zeros((DH - D_HID, D_OUT), jnp.float32)
    wmu = jnp.concatenate([wmu_ref[...], zpad], axis=0)
    wls = jnp.concatenate([wls_ref[...], zpad], axis=0)
    t = dinv_ref[...] * (q_ref[0] + q_ref[1] - hs2_ref[...])
    mu_ref[...] = jnp.dot(t, wmu,
                          preferred_element_type=jnp.float32) + bmu_ref[...]
    ls_ref[...] = jnp.dot(t, wls,
                          preferred_element_type=jnp.float32) + bls_ref[...]


_fin = pl.pallas_call(
    _fin_body,
    out_shape=(
        jax.ShapeDtypeStruct((N, D_OUT), jnp.float32),
        jax.ShapeDtypeStruct((N, D_OUT), jnp.float32),
    ),
)


def kernel(x, edge_index, W1, b1, Wmu, bmu, Wls, bls):
    row = edge_index[0].reshape(NC, NS, NCHUNK, CHUNK)
    col = edge_index[1].reshape(NC, NS, NCHUNK, CHUNK)
    ones = jnp.ones((CHUNK,), jnp.float32)
    zeros = jnp.zeros((624,), jnp.float32)

    degp = _deg_kernel(col, ones, zeros).reshape(NC, N)
    y1 = _mm1(x, W1)
    hs1, dinv = _scale(degp, y1)

    p = _agg_kernel(hs1, row, col)
    hs2 = _mid(p, hs1, dinv, b1.reshape(1, D_HID))
    q = _agg_kernel(hs2, row, col)
    mu, ls = _fin(q, hs2, dinv, Wmu, bmu.reshape(1, D_OUT),
                  Wls, bls.reshape(1, D_OUT))
    return (mu, ls)


# speedup vs baseline: 60.9827x; 1.0161x over previous
import functools
import jax
import jax.numpy as jnp
from jax import lax
from jax.experimental import pallas as pl
from jax.experimental.pallas import tpu as pltpu
from jax.experimental.pallas import tpu_sc as plsc

N = 10000
E = 320000
D_IN = 128
D_HID = 20
D_OUT = 10
DH = 32

NC, NS = 2, 16
NW = NC * NS
EPW = E // NW
CHUNK = 125
NCHUNK = EPW // CHUNK
ROWS_PER_SUB = N // NS

_mesh = plsc.VectorSubcoreMesh(
    core_axis_name="c", subcore_axis_name="s", num_cores=NC, num_subcores=NS
)


DCH = 128
NDCH = EPW // DCH
DTAIL = EPW - NDCH * DCH


@functools.partial(
    pl.kernel,
    out_type=jax.ShapeDtypeStruct((NC * N,), jnp.float32),
    mesh=_mesh,
    scratch_types=[
        pltpu.VMEM((EPW,), jnp.int32),
        pltpu.VMEM((DCH,), jnp.float32),
        pltpu.VMEM((624,), jnp.float32),
        pltpu.VMEM_SHARED((N,), jnp.float32),
        pltpu.SemaphoreType.DMA,
        pltpu.SemaphoreType.DMA,
    ],
)
def _deg_kernel(col_hbm, ones_hbm, zeros_hbm, out_hbm,
                colv, onesv, zbuf, acc, ci, ss):
    c = lax.axis_index("c")
    s = lax.axis_index("s")
    wid = c * NS + s
    M = 624
    REM = N - NS * M
    sl = pl.ds(s * M, M)
    rem = pl.ds(NS * M, REM)
    pltpu.async_copy(col_hbm.at[pl.ds(wid * EPW, EPW)], colv, ci)
    pltpu.sync_copy(zeros_hbm, zbuf)
    pltpu.sync_copy(zbuf, acc.at[sl])

    @pl.when(s == 0)
    def _():
        pltpu.sync_copy(zbuf.at[pl.ds(0, REM)], acc.at[rem])

    pltpu.sync_copy(ones_hbm, onesv)
    pltpu.make_async_copy(col_hbm.at[pl.ds(wid * EPW, EPW)], colv, ci).wait()
    plsc.subcore_barrier()

    def body(j, carry):
        pltpu.async_copy(onesv, acc.at[colv.at[pl.ds(j * DCH, DCH)]], ss,
                         add=True)
        return carry

    lax.fori_loop(0, NDCH, body, 0)
    pltpu.async_copy(onesv.at[pl.ds(0, DTAIL)],
                     acc.at[colv.at[pl.ds(NDCH * DCH, DTAIL)]], ss, add=True)

    def drain(j, carry):
        pltpu.make_async_copy(
            onesv, acc.at[colv.at[pl.ds(j * DCH, DCH)]], ss).wait()
        return carry

    lax.fori_loop(0, NDCH, drain, 0)
    pltpu.make_async_copy(onesv.at[pl.ds(0, DTAIL)],
                          acc.at[colv.at[pl.ds(NDCH * DCH, DTAIL)]], ss).wait()
    plsc.subcore_barrier()
    pltpu.sync_copy(acc.at[sl], zbuf)
    pltpu.sync_copy(zbuf, out_hbm.at[pl.ds(c * N + s * M, M)])

    @pl.when(s == 0)
    def _():
        pltpu.sync_copy(acc.at[rem], zbuf.at[pl.ds(0, REM)])
        pltpu.sync_copy(zbuf.at[pl.ds(0, REM)], out_hbm.at[pl.ds(c * N + NS * M, REM)])


@functools.partial(
    pl.kernel,
    out_type=jax.ShapeDtypeStruct((NC, N, DH), jnp.float32),
    mesh=_mesh,
    scratch_types=[
        pltpu.VMEM((NCHUNK, CHUNK), jnp.int32),
        pltpu.VMEM((NCHUNK, CHUNK), jnp.int32),
        pltpu.VMEM((CHUNK, DH), jnp.float32),
        pltpu.VMEM((CHUNK, DH), jnp.float32),
        pltpu.VMEM((CHUNK, DH), jnp.float32),
        pltpu.VMEM((CHUNK, DH), jnp.float32),
        pltpu.VMEM((624, DH), jnp.float32),
        pltpu.VMEM_SHARED((N, DH), jnp.float32),
        pltpu.SemaphoreType.DMA,
        pltpu.SemaphoreType.DMA,
        pltpu.SemaphoreType.DMA,
        pltpu.SemaphoreType.DMA,
        pltpu.SemaphoreType.DMA,
        pltpu.SemaphoreType.DMA,
        pltpu.SemaphoreType.DMA,
        pltpu.SemaphoreType.DMA,
        pltpu.SemaphoreType.DMA,
        pltpu.SemaphoreType.DMA,
    ],
    compiler_params=pltpu.CompilerParams(use_tc_tiling_on_sc=False),
)
def _agg_kernel(hs_hbm, row_hbm, col_hbm, out_hbm,
                rowv, colv, b0, b1, b2, b3, sbuf, acc,
                g0, g1, g2, g3, s0, s1, s2, s3, ri, ci):
    c = lax.axis_index("c")
    s = lax.axis_index("s")
    M = 624
    REM = N - NS * M
    sl = pl.ds(s * M, M)
    rem = pl.ds(NS * M, REM)
    bufs = (b0, b1, b2, b3)
    gsem = (g0, g1, g2, g3)
    ssem = (s0, s1, s2, s3)
    pltpu.async_copy(row_hbm.at[c, s], rowv, ri)
    pltpu.async_copy(col_hbm.at[c, s], colv, ci)
    pltpu.sync_copy(hs_hbm.at[sl], sbuf)
    pltpu.sync_copy(sbuf, acc.at[sl])

    @pl.when(s == 0)
    def _():
        pltpu.sync_copy(hs_hbm.at[rem], sbuf.at[pl.ds(0, REM)])
        pltpu.sync_copy(sbuf.at[pl.ds(0, REM)], acc.at[rem])

    pltpu.make_async_copy(row_hbm.at[c, s], rowv, ri).wait()
    pltpu.make_async_copy(col_hbm.at[c, s], colv, ci).wait()
    plsc.subcore_barrier()

    pltpu.async_copy(hs_hbm.at[rowv.at[0]], bufs[0], gsem[0])
    pltpu.async_copy(hs_hbm.at[rowv.at[1]], bufs[1], gsem[1])

    def body(jj, carry):
        j0 = jj * 4
        for b in range(4):
            j = j0 + b
            pltpu.make_async_copy(hs_hbm.at[rowv.at[j]], bufs[b], gsem[b]).wait()
            pltpu.async_copy(bufs[b], acc.at[colv.at[j]], ssem[b], add=True)
            m = j + 2
            mb = (b + 2) % 4

            @pl.when(m < NCHUNK)
            def _():
                @pl.when(m >= 4)
                def _():
                    pltpu.make_async_copy(
                        bufs[mb], acc.at[colv.at[m - 4]], ssem[mb]
                    ).wait()

                pltpu.async_copy(hs_hbm.at[rowv.at[m]], bufs[mb], gsem[mb])

        return carry

    lax.fori_loop(0, NCHUNK // 4, body, 0)
    for b in range(4):
        j = NCHUNK - 4 + b
        pltpu.make_async_copy(bufs[b], acc.at[colv.at[j]], ssem[b]).wait()
    plsc.subcore_barrier()
    pltpu.sync_copy(acc.at[sl], sbuf)
    pltpu.sync_copy(sbuf, out_hbm.at[c, sl])

    @pl.when(s == 0)
    def _():
        pltpu.sync_copy(acc.at[rem], sbuf.at[pl.ds(0, REM)])
        pltpu.sync_copy(sbuf.at[pl.ds(0, REM)], out_hbm.at[c, rem])


def _mm1_body(x_ref, w_ref, o_ref):
    w = jnp.concatenate(
        [w_ref[...], jnp.zeros((D_IN, DH - D_HID), jnp.float32)], axis=1)
    o_ref[...] = jnp.dot(x_ref[...], w, preferred_element_type=jnp.float32)


_mm1 = pl.pallas_call(
    _mm1_body,
    out_shape=jax.ShapeDtypeStruct((N, DH), jnp.float32),
)


def _scale_body(deg_ref, y_ref, hs_ref, dinv_ref):
    d = lax.rsqrt(deg_ref[0] + deg_ref[1] + 1.0).reshape(N, 1)
    dinv_ref[...] = d
    hs_ref[...] = d * y_ref[...]


_scale = pl.pallas_call(
    _scale_body,
    out_shape=(
        jax.ShapeDtypeStruct((N, DH), jnp.float32),
        jax.ShapeDtypeStruct((N, 1), jnp.float32),
    ),
)


def _mid_body(p_ref, hs1_ref, dinv_ref, b_ref, o_ref):
    b = jnp.concatenate(
        [b_ref[...], jnp.zeros((1, DH - D_HID), jnp.float32)], axis=1)
    agg = p_ref[0] + p_ref[1] - hs1_ref[...]
    h = jnp.maximum(dinv_ref[...] * agg + b, 0.0)
    o_ref[...] = dinv_ref[...] * h


_mid = pl.pallas_call(
    _mid_body,
    out_shape=jax.ShapeDtypeStruct((N, DH), jnp.float32),
)


def _fin_body(q_ref, hs2_ref, dinv_ref, wmu_ref, bmu_ref, wls_ref, bls_ref,
              mu_ref, ls_ref):
    zpad = jnp.zeros((DH - D_HID, D_OUT), jnp.float32)
    wmu = jnp.concatenate([wmu_ref[...], zpad], axis=0)
    wls = jnp.concatenate([wls_ref[...], zpad], axis=0)
    t = dinv_ref[...] * (q_ref[0] + q_ref[1] - hs2_ref[...])
    mu_ref[...] = jnp.dot(t, wmu,
                          preferred_element_type=jnp.float32) + bmu_ref[...]
    ls_ref[...] = jnp.dot(t, wls,
                          preferred_element_type=jnp.float32) + bls_ref[...]


_fin = pl.pallas_call(
    _fin_body,
    out_shape=(
        jax.ShapeDtypeStruct((N, D_OUT), jnp.float32),
        jax.ShapeDtypeStruct((N, D_OUT), jnp.float32),
    ),
)


def kernel(x, edge_index, W1, b1, Wmu, bmu, Wls, bls):
    row = edge_index[0].reshape(NC, NS, NCHUNK, CHUNK)
    col = edge_index[1].reshape(NC, NS, NCHUNK, CHUNK)
    col1d = edge_index[1]
    ones = jnp.ones((DCH,), jnp.float32)
    zeros = jnp.zeros((624,), jnp.float32)

    degp = _deg_kernel(col1d, ones, zeros).reshape(NC, N)
    y1 = _mm1(x, W1)
    hs1, dinv = _scale(degp, y1)

    p = _agg_kernel(hs1, row, col)
    hs2 = _mid(p, hs1, dinv, b1.reshape(1, D_HID))
    q = _agg_kernel(hs2, row, col)
    mu, ls = _fin(q, hs2, dinv, Wmu, bmu.reshape(1, D_OUT),
                  Wls, bls.reshape(1, D_OUT))
    return (mu, ls)


# speedup vs baseline: 63.6869x; 1.0443x over previous
import functools
import jax
import jax.numpy as jnp
from jax import lax
from jax.experimental import pallas as pl
from jax.experimental.pallas import tpu as pltpu
from jax.experimental.pallas import tpu_sc as plsc

N = 10000
E = 320000
D_IN = 128
D_HID = 20
D_OUT = 10
DH = 32

NC, NS = 2, 16
NW = NC * NS
EPW = E // NW
CHUNK = 125
NCHUNK = EPW // CHUNK
ROWS_PER_SUB = N // NS

_mesh = plsc.VectorSubcoreMesh(
    core_axis_name="c", subcore_axis_name="s", num_cores=NC, num_subcores=NS
)


DCH = 128
NDCH = EPW // DCH
DTAIL = EPW - NDCH * DCH


@functools.partial(
    pl.kernel,
    out_type=jax.ShapeDtypeStruct((NC * N,), jnp.float32),
    mesh=_mesh,
    scratch_types=[
        pltpu.VMEM((EPW,), jnp.int32),
        pltpu.VMEM((DCH,), jnp.float32),
        pltpu.VMEM((624,), jnp.float32),
        pltpu.VMEM_SHARED((N,), jnp.float32),
        pltpu.SemaphoreType.DMA,
        pltpu.SemaphoreType.DMA,
    ],
)
def _deg_kernel(col_hbm, ones_hbm, zeros_hbm, out_hbm,
                colv, onesv, zbuf, acc, ci, ss):
    c = lax.axis_index("c")
    s = lax.axis_index("s")
    wid = c * NS + s
    M = 624
    REM = N - NS * M
    sl = pl.ds(s * M, M)
    rem = pl.ds(NS * M, REM)
    pltpu.async_copy(col_hbm.at[pl.ds(wid * EPW, EPW)], colv, ci)
    pltpu.sync_copy(zeros_hbm, zbuf)
    pltpu.sync_copy(zbuf, acc.at[sl])

    @pl.when(s == 0)
    def _():
        pltpu.sync_copy(zbuf.at[pl.ds(0, REM)], acc.at[rem])

    pltpu.sync_copy(ones_hbm, onesv)
    pltpu.make_async_copy(col_hbm.at[pl.ds(wid * EPW, EPW)], colv, ci).wait()
    plsc.subcore_barrier()

    def body(j, carry):
        pltpu.async_copy(onesv, acc.at[colv.at[pl.ds(j * DCH, DCH)]], ss,
                         add=True)
        return carry

    lax.fori_loop(0, NDCH, body, 0)
    pltpu.async_copy(onesv.at[pl.ds(0, DTAIL)],
                     acc.at[colv.at[pl.ds(NDCH * DCH, DTAIL)]], ss, add=True)

    def drain(j, carry):
        pltpu.make_async_copy(
            onesv, acc.at[colv.at[pl.ds(j * DCH, DCH)]], ss).wait()
        return carry

    lax.fori_loop(0, NDCH, drain, 0)
    pltpu.make_async_copy(onesv.at[pl.ds(0, DTAIL)],
                          acc.at[colv.at[pl.ds(NDCH * DCH, DTAIL)]], ss).wait()
    plsc.subcore_barrier()
    pltpu.sync_copy(acc.at[sl], zbuf)
    pltpu.sync_copy(zbuf, out_hbm.at[pl.ds(c * N + s * M, M)])

    @pl.when(s == 0)
    def _():
        pltpu.sync_copy(acc.at[rem], zbuf.at[pl.ds(0, REM)])
        pltpu.sync_copy(zbuf.at[pl.ds(0, REM)], out_hbm.at[pl.ds(c * N + NS * M, REM)])


@functools.partial(
    pl.kernel,
    out_type=jax.ShapeDtypeStruct((NC, N, DH), jnp.float32),
    mesh=_mesh,
    scratch_types=[
        pltpu.VMEM((NCHUNK, CHUNK), jnp.int32),
        pltpu.VMEM((NCHUNK, CHUNK), jnp.int32),
        pltpu.VMEM((CHUNK, DH), jnp.float32),
        pltpu.VMEM((CHUNK, DH), jnp.float32),
        pltpu.VMEM((CHUNK, DH), jnp.float32),
        pltpu.VMEM((CHUNK, DH), jnp.float32),
        pltpu.VMEM((624, DH), jnp.float32),
        pltpu.VMEM_SHARED((N, DH), jnp.float32),
        pltpu.SemaphoreType.DMA,
        pltpu.SemaphoreType.DMA,
        pltpu.SemaphoreType.DMA,
        pltpu.SemaphoreType.DMA,
        pltpu.SemaphoreType.DMA,
        pltpu.SemaphoreType.DMA,
        pltpu.SemaphoreType.DMA,
        pltpu.SemaphoreType.DMA,
        pltpu.SemaphoreType.DMA,
        pltpu.SemaphoreType.DMA,
    ],
    compiler_params=pltpu.CompilerParams(use_tc_tiling_on_sc=False),
)
def _agg_kernel(hs_hbm, row_hbm, col_hbm, out_hbm,
                rowv, colv, b0, b1, b2, b3, sbuf, acc,
                g0, g1, g2, g3, s0, s1, s2, s3, ri, ci):
    c = lax.axis_index("c")
    s = lax.axis_index("s")
    M = 624
    REM = N - NS * M
    sl = pl.ds(s * M, M)
    rem = pl.ds(NS * M, REM)
    bufs = (b0, b1, b2, b3)
    gsem = (g0, g1, g2, g3)
    ssem = (s0, s1, s2, s3)
    pltpu.async_copy(row_hbm.at[c, s], rowv, ri)
    pltpu.async_copy(col_hbm.at[c, s], colv, ci)
    pltpu.sync_copy(hs_hbm.at[sl], sbuf)
    pltpu.sync_copy(sbuf, acc.at[sl])

    @pl.when(s == 0)
    def _():
        pltpu.sync_copy(hs_hbm.at[rem], sbuf.at[pl.ds(0, REM)])
        pltpu.sync_copy(sbuf.at[pl.ds(0, REM)], acc.at[rem])

    pltpu.make_async_copy(row_hbm.at[c, s], rowv, ri).wait()
    pltpu.make_async_copy(col_hbm.at[c, s], colv, ci).wait()
    plsc.subcore_barrier()

    pltpu.async_copy(hs_hbm.at[rowv.at[0]], bufs[0], gsem[0])
    pltpu.async_copy(hs_hbm.at[rowv.at[1]], bufs[1], gsem[1])

    def body(jj, carry):
        j0 = jj * 4
        for b in range(4):
            j = j0 + b
            pltpu.make_async_copy(hs_hbm.at[rowv.at[j]], bufs[b], gsem[b]).wait()
            pltpu.async_copy(bufs[b], acc.at[colv.at[j]], ssem[b], add=True)
            m = j + 2
            mb = (b + 2) % 4

            @pl.when(m < NCHUNK)
            def _():
                @pl.when(m >= 4)
                def _():
                    pltpu.make_async_copy(
                        bufs[mb], acc.at[colv.at[m - 4]], ssem[mb]
                    ).wait()

                pltpu.async_copy(hs_hbm.at[rowv.at[m]], bufs[mb], gsem[mb])

        return carry

    lax.fori_loop(0, NCHUNK // 4, body, 0)
    for b in range(4):
        j = NCHUNK - 4 + b
        pltpu.make_async_copy(bufs[b], acc.at[colv.at[j]], ssem[b]).wait()
    plsc.subcore_barrier()
    pltpu.sync_copy(acc.at[sl], sbuf)
    pltpu.sync_copy(sbuf, out_hbm.at[c, sl])

    @pl.when(s == 0)
    def _():
        pltpu.sync_copy(acc.at[rem], sbuf.at[pl.ds(0, REM)])
        pltpu.sync_copy(sbuf.at[pl.ds(0, REM)], out_hbm.at[c, rem])


MROWS = 312
MTAIL = N - NW * MROWS


@functools.partial(
    pl.kernel,
    out_type=jax.ShapeDtypeStruct((N, DH), jnp.float32),
    mesh=_mesh,
    scratch_types=[
        pltpu.VMEM((MROWS + MTAIL, DH), jnp.float32),
        pltpu.VMEM((MROWS + MTAIL, DH), jnp.float32),
        pltpu.VMEM((MROWS + MTAIL, DH), jnp.float32),
        pltpu.VMEM((MROWS + MTAIL + 16,), jnp.float32),
        pltpu.VMEM((DH,), jnp.float32),
    ],
    compiler_params=pltpu.CompilerParams(use_tc_tiling_on_sc=False),
)
def _mid_kernel(p_hbm, hs1_hbm, dinv_hbm, b_hbm, out_hbm, t0, t1, t2, td, tb):
    c = lax.axis_index("c")
    s = lax.axis_index("s")
    wid = c * NS + s
    base = wid * MROWS
    sl = pl.ds(base, MROWS)
    pltpu.sync_copy(p_hbm.at[0, sl], t0.at[pl.ds(0, MROWS)])
    pltpu.sync_copy(p_hbm.at[1, sl], t1.at[pl.ds(0, MROWS)])
    pltpu.sync_copy(hs1_hbm.at[sl], t2.at[pl.ds(0, MROWS)])
    pltpu.sync_copy(dinv_hbm.at[pl.ds(base, MROWS)], td.at[pl.ds(0, MROWS)])
    pltpu.sync_copy(b_hbm, tb)

    @pl.when(wid == NW - 1)
    def _():
        tl = pl.ds(N - MTAIL, MTAIL)
        pltpu.sync_copy(p_hbm.at[0, tl], t0.at[pl.ds(MROWS, MTAIL)])
        pltpu.sync_copy(p_hbm.at[1, tl], t1.at[pl.ds(MROWS, MTAIL)])
        pltpu.sync_copy(hs1_hbm.at[tl], t2.at[pl.ds(MROWS, MTAIL)])
        pltpu.sync_copy(dinv_hbm.at[pl.ds(N - MTAIL, MTAIL)],
                        td.at[pl.ds(MROWS, MTAIL)])

    blo = tb[0:16]
    bhi = tb[16:DH]

    def body(r, carry):
        d = td[pl.ds(r, 16)][0]
        for h, bv in ((0, blo), (16, bhi)):
            agg = t0[r, h:h + 16] + t1[r, h:h + 16] - t2[r, h:h + 16]
            hval = jnp.maximum(d * agg + bv, 0.0)
            t0[r, h:h + 16] = d * hval
        return carry

    lax.fori_loop(0, MROWS, body, 0)

    @pl.when(wid == NW - 1)
    def _():
        lax.fori_loop(MROWS, MROWS + MTAIL, body, 0)

    pltpu.sync_copy(t0.at[pl.ds(0, MROWS)], out_hbm.at[sl])

    @pl.when(wid == NW - 1)
    def _():
        pltpu.sync_copy(t0.at[pl.ds(MROWS, MTAIL)],
                        out_hbm.at[pl.ds(N - MTAIL, MTAIL)])


def _mm1_body(x_ref, w_ref, o_ref):
    w = jnp.concatenate(
        [w_ref[...], jnp.zeros((D_IN, DH - D_HID), jnp.float32)], axis=1)
    o_ref[...] = jnp.dot(x_ref[...], w, preferred_element_type=jnp.float32)


_mm1 = pl.pallas_call(
    _mm1_body,
    out_shape=jax.ShapeDtypeStruct((N, DH), jnp.float32),
)


def _scale_body(deg_ref, y_ref, hs_ref, dinv_ref, dinv1_ref):
    d = lax.rsqrt(deg_ref[0] + deg_ref[1] + 1.0).reshape(N, 1)
    dinv_ref[...] = d
    dinv1_ref[...] = d.reshape(N)
    hs_ref[...] = d * y_ref[...]


_scale = pl.pallas_call(
    _scale_body,
    out_shape=(
        jax.ShapeDtypeStruct((N, DH), jnp.float32),
        jax.ShapeDtypeStruct((N, 1), jnp.float32),
        jax.ShapeDtypeStruct((N,), jnp.float32),
    ),
)


def _fin_body(q_ref, hs2_ref, dinv_ref, wmu_ref, bmu_ref, wls_ref, bls_ref,
              mu_ref, ls_ref):
    zpad = jnp.zeros((DH - D_HID, D_OUT), jnp.float32)
    wmu = jnp.concatenate([wmu_ref[...], zpad], axis=0)
    wls = jnp.concatenate([wls_ref[...], zpad], axis=0)
    t = dinv_ref[...] * (q_ref[0] + q_ref[1] - hs2_ref[...])
    mu_ref[...] = jnp.dot(t, wmu,
                          preferred_element_type=jnp.float32) + bmu_ref[...]
    ls_ref[...] = jnp.dot(t, wls,
                          preferred_element_type=jnp.float32) + bls_ref[...]


_fin = pl.pallas_call(
    _fin_body,
    out_shape=(
        jax.ShapeDtypeStruct((N, D_OUT), jnp.float32),
        jax.ShapeDtypeStruct((N, D_OUT), jnp.float32),
    ),
)


def kernel(x, edge_index, W1, b1, Wmu, bmu, Wls, bls):
    row = edge_index[0].reshape(NC, NS, NCHUNK, CHUNK)
    col = edge_index[1].reshape(NC, NS, NCHUNK, CHUNK)
    col1d = edge_index[1]
    ones = jnp.ones((DCH,), jnp.float32)
    zeros = jnp.zeros((624,), jnp.float32)

    degp = _deg_kernel(col1d, ones, zeros).reshape(NC, N)
    y1 = _mm1(x, W1)
    hs1, dinv, dinv1 = _scale(degp, y1)
    b32 = jnp.zeros((DH,), jnp.float32).at[:D_HID].set(b1)

    p = _agg_kernel(hs1, row, col)
    hs2 = _mid_kernel(p, hs1, dinv1, b32)
    q = _agg_kernel(hs2, row, col)
    mu, ls = _fin(q, hs2, dinv, Wmu, bmu.reshape(1, D_OUT),
                  Wls, bls.reshape(1, D_OUT))
    return (mu, ls)


# speedup vs baseline: 70.9274x; 1.1137x over previous
import functools
import jax
import jax.numpy as jnp
from jax import lax
from jax.experimental import pallas as pl
from jax.experimental.pallas import tpu as pltpu
from jax.experimental.pallas import tpu_sc as plsc

N = 10000
E = 320000
D_IN = 128
D_HID = 20
D_OUT = 10
DH = 32

NC, NS = 2, 16
NW = NC * NS
EPW = E // NW
CHUNK = 125
NCHUNK = EPW // CHUNK
ROWS_PER_SUB = N // NS

_mesh = plsc.VectorSubcoreMesh(
    core_axis_name="c", subcore_axis_name="s", num_cores=NC, num_subcores=NS
)


DCH = 128
NDCH = EPW // DCH
DTAIL = EPW - NDCH * DCH


@functools.partial(
    pl.kernel,
    out_type=jax.ShapeDtypeStruct((NC * N,), jnp.float32),
    mesh=_mesh,
    scratch_types=[
        pltpu.VMEM((EPW,), jnp.int32),
        pltpu.VMEM((DCH,), jnp.float32),
        pltpu.VMEM((624,), jnp.float32),
        pltpu.VMEM_SHARED((N,), jnp.float32),
        pltpu.SemaphoreType.DMA,
        pltpu.SemaphoreType.DMA,
    ],
)
def _deg_kernel(col_hbm, ones_hbm, zeros_hbm, out_hbm,
                colv, onesv, zbuf, acc, ci, ss):
    c = lax.axis_index("c")
    s = lax.axis_index("s")
    wid = c * NS + s
    M = 624
    REM = N - NS * M
    sl = pl.ds(s * M, M)
    rem = pl.ds(NS * M, REM)
    pltpu.async_copy(col_hbm.at[pl.ds(wid * EPW, EPW)], colv, ci)
    pltpu.sync_copy(zeros_hbm, zbuf)
    pltpu.sync_copy(zbuf, acc.at[sl])

    @pl.when(s == 0)
    def _():
        pltpu.sync_copy(zbuf.at[pl.ds(0, REM)], acc.at[rem])

    pltpu.sync_copy(ones_hbm, onesv)
    pltpu.make_async_copy(col_hbm.at[pl.ds(wid * EPW, EPW)], colv, ci).wait()
    plsc.subcore_barrier()

    def body(j, carry):
        pltpu.async_copy(onesv, acc.at[colv.at[pl.ds(j * DCH, DCH)]], ss,
                         add=True)
        return carry

    lax.fori_loop(0, NDCH, body, 0)
    pltpu.async_copy(onesv.at[pl.ds(0, DTAIL)],
                     acc.at[colv.at[pl.ds(NDCH * DCH, DTAIL)]], ss, add=True)

    def drain(j, carry):
        pltpu.make_async_copy(
            onesv, acc.at[colv.at[pl.ds(j * DCH, DCH)]], ss).wait()
        return carry

    lax.fori_loop(0, NDCH, drain, 0)
    pltpu.make_async_copy(onesv.at[pl.ds(0, DTAIL)],
                          acc.at[colv.at[pl.ds(NDCH * DCH, DTAIL)]], ss).wait()
    plsc.subcore_barrier()
    pltpu.sync_copy(acc.at[sl], zbuf)
    pltpu.sync_copy(zbuf, out_hbm.at[pl.ds(c * N + s * M, M)])

    @pl.when(s == 0)
    def _():
        pltpu.sync_copy(acc.at[rem], zbuf.at[pl.ds(0, REM)])
        pltpu.sync_copy(zbuf.at[pl.ds(0, REM)], out_hbm.at[pl.ds(c * N + NS * M, REM)])


@functools.partial(
    pl.kernel,
    out_type=jax.ShapeDtypeStruct((NC, N, DH), jnp.float32),
    mesh=_mesh,
    scratch_types=[
        pltpu.VMEM((NCHUNK, CHUNK), jnp.int32),
        pltpu.VMEM((NCHUNK, CHUNK), jnp.int32),
        pltpu.VMEM((CHUNK, DH), jnp.float32),
        pltpu.VMEM((CHUNK, DH), jnp.float32),
        pltpu.VMEM((CHUNK, DH), jnp.float32),
        pltpu.VMEM((CHUNK, DH), jnp.float32),
        pltpu.VMEM((CHUNK, DH), jnp.float32),
        pltpu.VMEM((CHUNK, DH), jnp.float32),
        pltpu.VMEM((CHUNK, DH), jnp.float32),
        pltpu.VMEM((CHUNK, DH), jnp.float32),
        pltpu.VMEM((624, DH), jnp.float32),
        pltpu.VMEM_SHARED((N, DH), jnp.float32),
        pltpu.SemaphoreType.DMA,
        pltpu.SemaphoreType.DMA,
        pltpu.SemaphoreType.DMA,
        pltpu.SemaphoreType.DMA,
        pltpu.SemaphoreType.DMA,
        pltpu.SemaphoreType.DMA,
        pltpu.SemaphoreType.DMA,
        pltpu.SemaphoreType.DMA,
        pltpu.SemaphoreType.DMA,
        pltpu.SemaphoreType.DMA,
        pltpu.SemaphoreType.DMA,
        pltpu.SemaphoreType.DMA,
        pltpu.SemaphoreType.DMA,
        pltpu.SemaphoreType.DMA,
        pltpu.SemaphoreType.DMA,
        pltpu.SemaphoreType.DMA,
        pltpu.SemaphoreType.DMA,
        pltpu.SemaphoreType.DMA,
    ],
    compiler_params=pltpu.CompilerParams(use_tc_tiling_on_sc=False),
)
def _agg_kernel(hs_hbm, row_hbm, col_hbm, out_hbm,
                rowv, colv, b0, b1, b2, b3, b4, b5, b6, b7, sbuf, acc,
                g0, g1, g2, g3, g4, g5, g6, g7,
                s0, s1, s2, s3, s4, s5, s6, s7, ri, ci):
    c = lax.axis_index("c")
    s = lax.axis_index("s")
    M = 624
    REM = N - NS * M
    sl = pl.ds(s * M, M)
    rem = pl.ds(NS * M, REM)
    bufs = (b0, b1, b2, b3, b4, b5, b6, b7)
    gsem = (g0, g1, g2, g3, g4, g5, g6, g7)
    ssem = (s0, s1, s2, s3, s4, s5, s6, s7)
    pltpu.async_copy(row_hbm.at[c, s], rowv, ri)
    pltpu.async_copy(col_hbm.at[c, s], colv, ci)
    pltpu.sync_copy(hs_hbm.at[sl], sbuf)
    pltpu.sync_copy(sbuf, acc.at[sl])

    @pl.when(s == 0)
    def _():
        pltpu.sync_copy(hs_hbm.at[rem], sbuf.at[pl.ds(0, REM)])
        pltpu.sync_copy(sbuf.at[pl.ds(0, REM)], acc.at[rem])

    pltpu.make_async_copy(row_hbm.at[c, s], rowv, ri).wait()
    pltpu.make_async_copy(col_hbm.at[c, s], colv, ci).wait()
    plsc.subcore_barrier()

    RING = 8
    AHEAD = 4
    for j in range(AHEAD):
        pltpu.async_copy(hs_hbm.at[rowv.at[j]], bufs[j], gsem[j])

    def body(jj, carry):
        j0 = jj * RING
        for b in range(RING):
            j = j0 + b
            pltpu.make_async_copy(hs_hbm.at[rowv.at[j]], bufs[b], gsem[b]).wait()
            pltpu.async_copy(bufs[b], acc.at[colv.at[j]], ssem[b], add=True)
            m = j + AHEAD
            mb = (b + AHEAD) % RING

            @pl.when(m < NCHUNK)
            def _():
                @pl.when(m >= RING)
                def _():
                    pltpu.make_async_copy(
                        bufs[mb], acc.at[colv.at[m - RING]], ssem[mb]
                    ).wait()

                pltpu.async_copy(hs_hbm.at[rowv.at[m]], bufs[mb], gsem[mb])

        return carry

    lax.fori_loop(0, NCHUNK // RING, body, 0)
    for b in range(RING):
        j = NCHUNK - RING + b
        pltpu.make_async_copy(bufs[b], acc.at[colv.at[j]], ssem[b]).wait()
    plsc.subcore_barrier()
    pltpu.sync_copy(acc.at[sl], sbuf)
    pltpu.sync_copy(sbuf, out_hbm.at[c, sl])

    @pl.when(s == 0)
    def _():
        pltpu.sync_copy(acc.at[rem], sbuf.at[pl.ds(0, REM)])
        pltpu.sync_copy(sbuf.at[pl.ds(0, REM)], out_hbm.at[c, rem])


MROWS = 312
MTAIL = N - NW * MROWS


@functools.partial(
    pl.kernel,
    out_type=jax.ShapeDtypeStruct((N, DH), jnp.float32),
    mesh=_mesh,
    scratch_types=[
        pltpu.VMEM((MROWS + MTAIL, DH), jnp.float32),
        pltpu.VMEM((MROWS + MTAIL, DH), jnp.float32),
        pltpu.VMEM((MROWS + MTAIL, DH), jnp.float32),
        pltpu.VMEM((MROWS + MTAIL + 16,), jnp.float32),
        pltpu.VMEM((DH,), jnp.float32),
    ],
    compiler_params=pltpu.CompilerParams(use_tc_tiling_on_sc=False),
)
def _mid_kernel(p_hbm, hs1_hbm, dinv_hbm, b_hbm, out_hbm, t0, t1, t2, td, tb):
    c = lax.axis_index("c")
    s = lax.axis_index("s")
    wid = c * NS + s
    base = wid * MROWS
    sl = pl.ds(base, MROWS)
    pltpu.sync_copy(p_hbm.at[0, sl], t0.at[pl.ds(0, MROWS)])
    pltpu.sync_copy(p_hbm.at[1, sl], t1.at[pl.ds(0, MROWS)])
    pltpu.sync_copy(hs1_hbm.at[sl], t2.at[pl.ds(0, MROWS)])
    pltpu.sync_copy(dinv_hbm.at[pl.ds(base, MROWS)], td.at[pl.ds(0, MROWS)])
    pltpu.sync_copy(b_hbm, tb)

    @pl.when(wid == NW - 1)
    def _():
        tl = pl.ds(N - MTAIL, MTAIL)
        pltpu.sync_copy(p_hbm.at[0, tl], t0.at[pl.ds(MROWS, MTAIL)])
        pltpu.sync_copy(p_hbm.at[1, tl], t1.at[pl.ds(MROWS, MTAIL)])
        pltpu.sync_copy(hs1_hbm.at[tl], t2.at[pl.ds(MROWS, MTAIL)])
        pltpu.sync_copy(dinv_hbm.at[pl.ds(N - MTAIL, MTAIL)],
                        td.at[pl.ds(MROWS, MTAIL)])

    blo = tb[0:16]
    bhi = tb[16:DH]

    def body(r, carry):
        d = td[pl.ds(r, 16)][0]
        for h, bv in ((0, blo), (16, bhi)):
            agg = t0[r, h:h + 16] + t1[r, h:h + 16] - t2[r, h:h + 16]
            hval = jnp.maximum(d * agg + bv, 0.0)
            t0[r, h:h + 16] = d * hval
        return carry

    lax.fori_loop(0, MROWS, body, 0)

    @pl.when(wid == NW - 1)
    def _():
        lax.fori_loop(MROWS, MROWS + MTAIL, body, 0)

    pltpu.sync_copy(t0.at[pl.ds(0, MROWS)], out_hbm.at[sl])

    @pl.when(wid == NW - 1)
    def _():
        pltpu.sync_copy(t0.at[pl.ds(MROWS, MTAIL)],
                        out_hbm.at[pl.ds(N - MTAIL, MTAIL)])


def _mm1_body(x_ref, w_ref, o_ref):
    w = jnp.concatenate(
        [w_ref[...], jnp.zeros((D_IN, DH - D_HID), jnp.float32)], axis=1)
    o_ref[...] = jnp.dot(x_ref[...], w, preferred_element_type=jnp.float32)


_mm1 = pl.pallas_call(
    _mm1_body,
    out_shape=jax.ShapeDtypeStruct((N, DH), jnp.float32),
)


def _scale_body(deg_ref, y_ref, hs_ref, dinv_ref, dinv1_ref):
    d = lax.rsqrt(deg_ref[0] + deg_ref[1] + 1.0).reshape(N, 1)
    dinv_ref[...] = d
    dinv1_ref[...] = d.reshape(N)
    hs_ref[...] = d * y_ref[...]


_scale = pl.pallas_call(
    _scale_body,
    out_shape=(
        jax.ShapeDtypeStruct((N, DH), jnp.float32),
        jax.ShapeDtypeStruct((N, 1), jnp.float32),
        jax.ShapeDtypeStruct((N,), jnp.float32),
    ),
)


def _fin_body(q_ref, hs2_ref, dinv_ref, wmu_ref, bmu_ref, wls_ref, bls_ref,
              mu_ref, ls_ref):
    zpad = jnp.zeros((DH - D_HID, D_OUT), jnp.float32)
    wmu = jnp.concatenate([wmu_ref[...], zpad], axis=0)
    wls = jnp.concatenate([wls_ref[...], zpad], axis=0)
    t = dinv_ref[...] * (q_ref[0] + q_ref[1] - hs2_ref[...])
    mu_ref[...] = jnp.dot(t, wmu,
                          preferred_element_type=jnp.float32) + bmu_ref[...]
    ls_ref[...] = jnp.dot(t, wls,
                          preferred_element_type=jnp.float32) + bls_ref[...]


_fin = pl.pallas_call(
    _fin_body,
    out_shape=(
        jax.ShapeDtypeStruct((N, D_OUT), jnp.float32),
        jax.ShapeDtypeStruct((N, D_OUT), jnp.float32),
    ),
)


def kernel(x, edge_index, W1, b1, Wmu, bmu, Wls, bls):
    row = edge_index[0].reshape(NC, NS, NCHUNK, CHUNK)
    col = edge_index[1].reshape(NC, NS, NCHUNK, CHUNK)
    col1d = edge_index[1]
    ones = jnp.ones((DCH,), jnp.float32)
    zeros = jnp.zeros((624,), jnp.float32)

    degp = _deg_kernel(col1d, ones, zeros).reshape(NC, N)
    y1 = _mm1(x, W1)
    hs1, dinv, dinv1 = _scale(degp, y1)
    b32 = jnp.zeros((DH,), jnp.float32).at[:D_HID].set(b1)

    p = _agg_kernel(hs1, row, col)
    hs2 = _mid_kernel(p, hs1, dinv1, b32)
    q = _agg_kernel(hs2, row, col)
    mu, ls = _fin(q, hs2, dinv, Wmu, bmu.reshape(1, D_OUT),
                  Wls, bls.reshape(1, D_OUT))
    return (mu, ls)


# speedup vs baseline: 74.1595x; 1.0456x over previous
import functools
import jax
import jax.numpy as jnp
from jax import lax
from jax.experimental import pallas as pl
from jax.experimental.pallas import tpu as pltpu
from jax.experimental.pallas import tpu_sc as plsc

N = 10000
E = 320000
D_IN = 128
D_HID = 20
D_OUT = 10
DH = 32

NC, NS = 2, 16
NW = NC * NS
EPW = E // NW
CHUNK = 125
NCHUNK = EPW // CHUNK
ROWS_PER_SUB = N // NS

_mesh = plsc.VectorSubcoreMesh(
    core_axis_name="c", subcore_axis_name="s", num_cores=NC, num_subcores=NS
)


DCH = 128
NDCH = EPW // DCH
DTAIL = EPW - NDCH * DCH


@functools.partial(
    pl.kernel,
    out_type=jax.ShapeDtypeStruct((NC * N,), jnp.float32),
    mesh=_mesh,
    scratch_types=[
        pltpu.VMEM((EPW,), jnp.int32),
        pltpu.VMEM((DCH,), jnp.float32),
        pltpu.VMEM((624,), jnp.float32),
        pltpu.VMEM_SHARED((N,), jnp.float32),
        pltpu.SemaphoreType.DMA,
        pltpu.SemaphoreType.DMA,
    ],
)
def _deg_kernel(col_hbm, ones_hbm, zeros_hbm, out_hbm,
                colv, onesv, zbuf, acc, ci, ss):
    c = lax.axis_index("c")
    s = lax.axis_index("s")
    wid = c * NS + s
    M = 624
    REM = N - NS * M
    sl = pl.ds(s * M, M)
    rem = pl.ds(NS * M, REM)
    pltpu.async_copy(col_hbm.at[pl.ds(wid * EPW, EPW)], colv, ci)
    pltpu.sync_copy(zeros_hbm, zbuf)
    pltpu.sync_copy(zbuf, acc.at[sl])

    @pl.when(s == 0)
    def _():
        pltpu.sync_copy(zbuf.at[pl.ds(0, REM)], acc.at[rem])

    pltpu.sync_copy(ones_hbm, onesv)
    pltpu.make_async_copy(col_hbm.at[pl.ds(wid * EPW, EPW)], colv, ci).wait()
    plsc.subcore_barrier()

    def body(j, carry):
        pltpu.async_copy(onesv, acc.at[colv.at[pl.ds(j * DCH, DCH)]], ss,
                         add=True)
        return carry

    lax.fori_loop(0, NDCH, body, 0)
    pltpu.async_copy(onesv.at[pl.ds(0, DTAIL)],
                     acc.at[colv.at[pl.ds(NDCH * DCH, DTAIL)]], ss, add=True)

    def drain(j, carry):
        pltpu.make_async_copy(
            onesv, acc.at[colv.at[pl.ds(j * DCH, DCH)]], ss).wait()
        return carry

    lax.fori_loop(0, NDCH, drain, 0)
    pltpu.make_async_copy(onesv.at[pl.ds(0, DTAIL)],
                          acc.at[colv.at[pl.ds(NDCH * DCH, DTAIL)]], ss).wait()
    plsc.subcore_barrier()
    pltpu.sync_copy(acc.at[sl], zbuf)
    pltpu.sync_copy(zbuf, out_hbm.at[pl.ds(c * N + s * M, M)])

    @pl.when(s == 0)
    def _():
        pltpu.sync_copy(acc.at[rem], zbuf.at[pl.ds(0, REM)])
        pltpu.sync_copy(zbuf.at[pl.ds(0, REM)], out_hbm.at[pl.ds(c * N + NS * M, REM)])


@functools.partial(
    pl.kernel,
    out_type=jax.ShapeDtypeStruct((NC, N, DH), jnp.float32),
    mesh=_mesh,
    scratch_types=[
        pltpu.VMEM((NCHUNK, CHUNK), jnp.int32),
        pltpu.VMEM((NCHUNK, CHUNK), jnp.int32),
        pltpu.VMEM((CHUNK, DH), jnp.float32),
        pltpu.VMEM((CHUNK, DH), jnp.float32),
        pltpu.VMEM((CHUNK, DH), jnp.float32),
        pltpu.VMEM((CHUNK, DH), jnp.float32),
        pltpu.VMEM((CHUNK, DH), jnp.float32),
        pltpu.VMEM((CHUNK, DH), jnp.float32),
        pltpu.VMEM((CHUNK, DH), jnp.float32),
        pltpu.VMEM((CHUNK, DH), jnp.float32),
        pltpu.VMEM((624, DH), jnp.float32),
        pltpu.VMEM_SHARED((N, DH), jnp.float32),
        pltpu.SemaphoreType.DMA,
        pltpu.SemaphoreType.DMA,
        pltpu.SemaphoreType.DMA,
        pltpu.SemaphoreType.DMA,
        pltpu.SemaphoreType.DMA,
        pltpu.SemaphoreType.DMA,
        pltpu.SemaphoreType.DMA,
        pltpu.SemaphoreType.DMA,
        pltpu.SemaphoreType.DMA,
        pltpu.SemaphoreType.DMA,
        pltpu.SemaphoreType.DMA,
        pltpu.SemaphoreType.DMA,
        pltpu.SemaphoreType.DMA,
        pltpu.SemaphoreType.DMA,
        pltpu.SemaphoreType.DMA,
        pltpu.SemaphoreType.DMA,
        pltpu.SemaphoreType.DMA,
        pltpu.SemaphoreType.DMA,
    ],
    compiler_params=pltpu.CompilerParams(use_tc_tiling_on_sc=False),
)
def _agg_kernel(hs_hbm, row_hbm, col_hbm, out_hbm,
                rowv, colv, b0, b1, b2, b3, b4, b5, b6, b7, sbuf, acc,
                g0, g1, g2, g3, g4, g5, g6, g7,
                s0, s1, s2, s3, s4, s5, s6, s7, ri, ci):
    c = lax.axis_index("c")
    s = lax.axis_index("s")
    M = 624
    REM = N - NS * M
    sl = pl.ds(s * M, M)
    rem = pl.ds(NS * M, REM)
    bufs = (b0, b1, b2, b3, b4, b5, b6, b7)
    gsem = (g0, g1, g2, g3, g4, g5, g6, g7)
    ssem = (s0, s1, s2, s3, s4, s5, s6, s7)
    pltpu.async_copy(row_hbm.at[c, s], rowv, ri)
    pltpu.async_copy(col_hbm.at[c, s], colv, ci)
    pltpu.sync_copy(hs_hbm.at[sl], sbuf)
    pltpu.sync_copy(sbuf, acc.at[sl])

    @pl.when(s == 0)
    def _():
        pltpu.sync_copy(hs_hbm.at[rem], sbuf.at[pl.ds(0, REM)])
        pltpu.sync_copy(sbuf.at[pl.ds(0, REM)], acc.at[rem])

    pltpu.make_async_copy(row_hbm.at[c, s], rowv, ri).wait()
    pltpu.make_async_copy(col_hbm.at[c, s], colv, ci).wait()
    plsc.subcore_barrier()

    RING = 8
    AHEAD = 6
    for j in range(AHEAD):
        pltpu.async_copy(hs_hbm.at[rowv.at[j]], bufs[j], gsem[j])

    def body(jj, carry):
        j0 = jj * RING
        for b in range(RING):
            j = j0 + b
            pltpu.make_async_copy(hs_hbm.at[rowv.at[j]], bufs[b], gsem[b]).wait()
            pltpu.async_copy(bufs[b], acc.at[colv.at[j]], ssem[b], add=True)
            m = j + AHEAD
            mb = (b + AHEAD) % RING

            @pl.when(m < NCHUNK)
            def _():
                @pl.when(m >= RING)
                def _():
                    pltpu.make_async_copy(
                        bufs[mb], acc.at[colv.at[m - RING]], ssem[mb]
                    ).wait()

                pltpu.async_copy(hs_hbm.at[rowv.at[m]], bufs[mb], gsem[mb])

        return carry

    lax.fori_loop(0, NCHUNK // RING, body, 0)
    for b in range(RING):
        j = NCHUNK - RING + b
        pltpu.make_async_copy(bufs[b], acc.at[colv.at[j]], ssem[b]).wait()
    plsc.subcore_barrier()
    pltpu.sync_copy(acc.at[sl], sbuf)
    pltpu.sync_copy(sbuf, out_hbm.at[c, sl])

    @pl.when(s == 0)
    def _():
        pltpu.sync_copy(acc.at[rem], sbuf.at[pl.ds(0, REM)])
        pltpu.sync_copy(sbuf.at[pl.ds(0, REM)], out_hbm.at[c, rem])


MROWS = 312
MTAIL = N - NW * MROWS


@functools.partial(
    pl.kernel,
    out_type=jax.ShapeDtypeStruct((N, DH), jnp.float32),
    mesh=_mesh,
    scratch_types=[
        pltpu.VMEM((MROWS + MTAIL, DH), jnp.float32),
        pltpu.VMEM((MROWS + MTAIL, DH), jnp.float32),
        pltpu.VMEM((MROWS + MTAIL, DH), jnp.float32),
        pltpu.VMEM((MROWS + MTAIL + 16,), jnp.float32),
        pltpu.VMEM((DH,), jnp.float32),
    ],
    compiler_params=pltpu.CompilerParams(use_tc_tiling_on_sc=False),
)
def _mid_kernel(p_hbm, hs1_hbm, dinv_hbm, b_hbm, out_hbm, t0, t1, t2, td, tb):
    c = lax.axis_index("c")
    s = lax.axis_index("s")
    wid = c * NS + s
    base = wid * MROWS
    sl = pl.ds(base, MROWS)
    pltpu.sync_copy(p_hbm.at[0, sl], t0.at[pl.ds(0, MROWS)])
    pltpu.sync_copy(p_hbm.at[1, sl], t1.at[pl.ds(0, MROWS)])
    pltpu.sync_copy(hs1_hbm.at[sl], t2.at[pl.ds(0, MROWS)])
    pltpu.sync_copy(dinv_hbm.at[pl.ds(base, MROWS)], td.at[pl.ds(0, MROWS)])
    pltpu.sync_copy(b_hbm, tb)

    @pl.when(wid == NW - 1)
    def _():
        tl = pl.ds(N - MTAIL, MTAIL)
        pltpu.sync_copy(p_hbm.at[0, tl], t0.at[pl.ds(MROWS, MTAIL)])
        pltpu.sync_copy(p_hbm.at[1, tl], t1.at[pl.ds(MROWS, MTAIL)])
        pltpu.sync_copy(hs1_hbm.at[tl], t2.at[pl.ds(MROWS, MTAIL)])
        pltpu.sync_copy(dinv_hbm.at[pl.ds(N - MTAIL, MTAIL)],
                        td.at[pl.ds(MROWS, MTAIL)])

    blo = tb[0:16]
    bhi = tb[16:DH]

    def body(r, carry):
        d = td[pl.ds(r, 16)][0]
        for h, bv in ((0, blo), (16, bhi)):
            agg = t0[r, h:h + 16] + t1[r, h:h + 16] - t2[r, h:h + 16]
            hval = jnp.maximum(d * agg + bv, 0.0)
            t0[r, h:h + 16] = d * hval
        return carry

    lax.fori_loop(0, MROWS, body, 0)

    @pl.when(wid == NW - 1)
    def _():
        lax.fori_loop(MROWS, MROWS + MTAIL, body, 0)

    pltpu.sync_copy(t0.at[pl.ds(0, MROWS)], out_hbm.at[sl])

    @pl.when(wid == NW - 1)
    def _():
        pltpu.sync_copy(t0.at[pl.ds(MROWS, MTAIL)],
                        out_hbm.at[pl.ds(N - MTAIL, MTAIL)])


def _mm1_body(x_ref, w_ref, o_ref):
    w = jnp.concatenate(
        [w_ref[...], jnp.zeros((D_IN, DH - D_HID), jnp.float32)], axis=1)
    o_ref[...] = jnp.dot(x_ref[...], w, preferred_element_type=jnp.float32)


_mm1 = pl.pallas_call(
    _mm1_body,
    out_shape=jax.ShapeDtypeStruct((N, DH), jnp.float32),
)


def _scale_body(deg_ref, y_ref, hs_ref, dinv_ref, dinv1_ref):
    d = lax.rsqrt(deg_ref[0] + deg_ref[1] + 1.0).reshape(N, 1)
    dinv_ref[...] = d
    dinv1_ref[...] = d.reshape(N)
    hs_ref[...] = d * y_ref[...]


_scale = pl.pallas_call(
    _scale_body,
    out_shape=(
        jax.ShapeDtypeStruct((N, DH), jnp.float32),
        jax.ShapeDtypeStruct((N, 1), jnp.float32),
        jax.ShapeDtypeStruct((N,), jnp.float32),
    ),
)


def _fin_body(q_ref, hs2_ref, dinv_ref, wmu_ref, bmu_ref, wls_ref, bls_ref,
              mu_ref, ls_ref):
    zpad = jnp.zeros((DH - D_HID, D_OUT), jnp.float32)
    wmu = jnp.concatenate([wmu_ref[...], zpad], axis=0)
    wls = jnp.concatenate([wls_ref[...], zpad], axis=0)
    t = dinv_ref[...] * (q_ref[0] + q_ref[1] - hs2_ref[...])
    mu_ref[...] = jnp.dot(t, wmu,
                          preferred_element_type=jnp.float32) + bmu_ref[...]
    ls_ref[...] = jnp.dot(t, wls,
                          preferred_element_type=jnp.float32) + bls_ref[...]


_fin = pl.pallas_call(
    _fin_body,
    out_shape=(
        jax.ShapeDtypeStruct((N, D_OUT), jnp.float32),
        jax.ShapeDtypeStruct((N, D_OUT), jnp.float32),
    ),
)


def kernel(x, edge_index, W1, b1, Wmu, bmu, Wls, bls):
    row = edge_index[0].reshape(NC, NS, NCHUNK, CHUNK)
    col = edge_index[1].reshape(NC, NS, NCHUNK, CHUNK)
    col1d = edge_index[1]
    ones = jnp.ones((DCH,), jnp.float32)
    zeros = jnp.zeros((624,), jnp.float32)

    degp = _deg_kernel(col1d, ones, zeros).reshape(NC, N)
    y1 = _mm1(x, W1)
    hs1, dinv, dinv1 = _scale(degp, y1)
    b32 = jnp.zeros((DH,), jnp.float32).at[:D_HID].set(b1)

    p = _agg_kernel(hs1, row, col)
    hs2 = _mid_kernel(p, hs1, dinv1, b32)
    q = _agg_kernel(hs2, row, col)
    mu, ls = _fin(q, hs2, dinv, Wmu, bmu.reshape(1, D_OUT),
                  Wls, bls.reshape(1, D_OUT))
    return (mu, ls)
